# baseline (speedup 1.0000x reference)
import functools

import jax
import jax.numpy as jnp
from jax import lax
from jax.experimental import pallas as pl
from jax.experimental.pallas import tpu as pltpu
from jax.experimental.pallas import tpu_sc as plsc

N_NODES = 10000
N_EDGES = 320000
D_FEAT = 128

_INFO = plsc.get_sparse_core_info()
_NC = _INFO.num_cores
_NS = _INFO.num_subcores
_NW = _NC * _NS
_L = _INFO.num_lanes

_C = 128
_NCHUNKS = N_EDGES // _C
_ROUNDS = -(-_NCHUNKS // _NW)


def _make_kernel():
    mesh = plsc.VectorSubcoreMesh(core_axis_name="c", subcore_axis_name="s")

    @functools.partial(
        pl.kernel,
        mesh=mesh,
        out_type=jax.ShapeDtypeStruct((N_EDGES,), jnp.float32),
        scratch_types=[
            pltpu.VMEM((_C,), jnp.int32),
            pltpu.VMEM((_C,), jnp.int32),
            pltpu.VMEM((_C, D_FEAT), jnp.float32),
            pltpu.VMEM((_C, D_FEAT), jnp.float32),
            pltpu.VMEM((_C,), jnp.float32),
            pltpu.SemaphoreType.DMA,
            pltpu.SemaphoreType.DMA,
        ],
        compiler_params=pltpu.CompilerParams(needs_layout_passes=False),
    )
    def dot_scores(src_hbm, dst_hbm, h_hbm, out_hbm,
                   idx_u, idx_v, u_rows, v_rows, scores, sem_u, sem_v):
        wid = lax.axis_index("s") * _NC + lax.axis_index("c")
        lanes = lax.iota(jnp.int32, _L)

        def chunk_body(t, carry):
            cid = t * _NW + wid

            @pl.when(cid < _NCHUNKS)
            def _():
                base = cid * _C
                pltpu.sync_copy(src_hbm.at[pl.ds(base, _C)], idx_u)
                pltpu.sync_copy(dst_hbm.at[pl.ds(base, _C)], idx_v)
                cu = pltpu.async_copy(h_hbm.at[idx_u], u_rows, sem_u)
                cv = pltpu.async_copy(h_hbm.at[idx_v], v_rows, sem_v)
                cu.wait()
                cv.wait()

                def group_body(g, gcarry):
                    rows = g * _L + lanes
                    a0 = jnp.zeros((_L,), jnp.float32)
                    a1 = jnp.zeros((_L,), jnp.float32)
                    a2 = jnp.zeros((_L,), jnp.float32)
                    a3 = jnp.zeros((_L,), jnp.float32)
                    accs = [a0, a1, a2, a3]
                    for d in range(D_FEAT):
                        col = jnp.full((_L,), d, jnp.int32)
                        uu = plsc.load_gather(u_rows, [rows, col])
                        vv = plsc.load_gather(v_rows, [rows, col])
                        accs[d % 4] = accs[d % 4] + uu * vv
                    acc = (accs[0] + accs[1]) + (accs[2] + accs[3])
                    plsc.store_scatter(scores, [rows], acc)
                    return gcarry

                lax.fori_loop(0, _C // _L, group_body, 0)
                pltpu.sync_copy(scores, out_hbm.at[pl.ds(base, _C)])

            return carry

        lax.fori_loop(0, _ROUNDS, chunk_body, 0)

    return dot_scores


_dot_scores = _make_kernel()


def kernel(h, edge_index):
    ei = edge_index.astype(jnp.int32)
    return _dot_scores(ei[0], ei[1], h)


# speedup vs baseline: 1.3419x; 1.3419x over previous
import functools

import jax
import jax.numpy as jnp
from jax import lax
from jax.experimental import pallas as pl
from jax.experimental.pallas import tpu as pltpu
from jax.experimental.pallas import tpu_sc as plsc

N_NODES = 10000
N_EDGES = 320000
D_FEAT = 128

_INFO = plsc.get_sparse_core_info()
_NC = _INFO.num_cores
_NS = _INFO.num_subcores
_NW = _NC * _NS
_L = _INFO.num_lanes

_C = 64
_NCHUNKS = N_EDGES // _C
_ROUNDS = -(-_NCHUNKS // _NW)
_DBLK = 16


def _make_kernel():
    mesh = plsc.VectorSubcoreMesh(core_axis_name="c", subcore_axis_name="s")

    @functools.partial(
        pl.kernel,
        mesh=mesh,
        out_type=jax.ShapeDtypeStruct((N_EDGES,), jnp.float32),
        scratch_types=[
            pltpu.VMEM((2, _C), jnp.int32),
            pltpu.VMEM((2, _C), jnp.int32),
            pltpu.VMEM((_C, D_FEAT), jnp.float32),
            pltpu.VMEM((_C, D_FEAT), jnp.float32),
            pltpu.VMEM((_C, D_FEAT), jnp.float32),
            pltpu.VMEM((_C, D_FEAT), jnp.float32),
            pltpu.VMEM((_C,), jnp.float32),
            pltpu.VMEM_SHARED((N_NODES, D_FEAT), jnp.float32),
            pltpu.SemaphoreType.DMA,
            pltpu.SemaphoreType.DMA,
            pltpu.SemaphoreType.DMA,
            pltpu.SemaphoreType.DMA,
        ],
        compiler_params=pltpu.CompilerParams(needs_layout_passes=False),
    )
    def dot_scores(idx_hbm, h_hbm, out_hbm,
                   i0, i1, u0, u1, v0, v1, scores, h_sp,
                   su0, su1, sv0, sv1):
        wid = lax.axis_index("s") * _NC + lax.axis_index("c")
        sid = lax.axis_index("s")
        lanes = lax.iota(jnp.int32, _L)

        idx_slot = [i0, i1]
        u_slot = [u0, u1]
        v_slot = [v0, v1]
        su_slot = [su0, su1]
        sv_slot = [sv0, sv1]

        rows_per_tile = 624
        stage0 = sid * rows_per_tile
        pltpu.sync_copy(h_hbm.at[pl.ds(stage0, rows_per_tile)],
                        h_sp.at[pl.ds(stage0, rows_per_tile)])

        @pl.when(sid == 0)
        def _():
            tail = N_NODES - _NS * rows_per_tile
            pltpu.sync_copy(h_hbm.at[pl.ds(_NS * rows_per_tile, tail)],
                            h_sp.at[pl.ds(_NS * rows_per_tile, tail)])

        plsc.subcore_barrier()

        def issue(t, slot):
            cid = t * _NW + wid

            @pl.when(cid < _NCHUNKS)
            def _():
                pltpu.sync_copy(idx_hbm.at[cid], idx_slot[slot])
                pltpu.async_copy(h_sp.at[idx_slot[slot].at[0]],
                                 u_slot[slot], su_slot[slot])
                pltpu.async_copy(h_sp.at[idx_slot[slot].at[1]],
                                 v_slot[slot], sv_slot[slot])

        def consume(t, slot):
            cid = t * _NW + wid

            @pl.when(cid < _NCHUNKS)
            def _():
                u_rows = u_slot[slot]
                v_rows = v_slot[slot]
                pltpu.make_async_copy(h_sp.at[idx_slot[slot].at[0]],
                                      u_rows, su_slot[slot]).wait()
                pltpu.make_async_copy(h_sp.at[idx_slot[slot].at[1]],
                                      v_rows, sv_slot[slot]).wait()

                def group_body(g, gcarry):
                    rows = g * _L + lanes

                    def d_block(b, accs):
                        a0, a1, a2, a3 = accs
                        dbase = b * _DBLK
                        news = [a0, a1, a2, a3]
                        for j in range(_DBLK):
                            col = jnp.full((_L,), dbase + j, jnp.int32)
                            uu = plsc.load_gather(u_rows, [rows, col])
                            vv = plsc.load_gather(v_rows, [rows, col])
                            news[j % 4] = news[j % 4] + uu * vv
                        return tuple(news)

                    z = jnp.zeros((_L,), jnp.float32)
                    a0, a1, a2, a3 = lax.fori_loop(
                        0, D_FEAT // _DBLK, d_block, (z, z, z, z))
                    acc = (a0 + a1) + (a2 + a3)
                    plsc.store_scatter(scores, [rows], acc)
                    return gcarry

                lax.fori_loop(0, _C // _L, group_body, 0)
                pltpu.sync_copy(scores, out_hbm.at[pl.ds(cid * _C, _C)])

        issue(jnp.int32(0), 0)

        def pair_body(p, carry):
            t0 = p * 2
            issue(t0 + 1, 1)
            consume(t0, 0)
            issue(t0 + 2, 0)
            consume(t0 + 1, 1)
            return carry

        lax.fori_loop(0, (_ROUNDS + 1) // 2, pair_body, 0)

    return dot_scores


_dot_scores = _make_kernel()


def kernel(h, edge_index):
    idx = edge_index.astype(jnp.int32).reshape(2, _NCHUNKS, _C)
    idx_packed = idx.transpose(1, 0, 2)
    return _dot_scores(idx_packed, h)


# speedup vs baseline: 4.6585x; 3.4715x over previous
import functools

import jax
import jax.numpy as jnp
from jax import lax
from jax.experimental import pallas as pl
from jax.experimental.pallas import tpu as pltpu
from jax.experimental.pallas import tpu_sc as plsc

N_NODES = 10000
N_EDGES = 320000
D_FEAT = 128

_INFO = plsc.get_sparse_core_info()
_NC = _INFO.num_cores
_NS = _INFO.num_subcores
_NW = _NC * _NS
_L = _INFO.num_lanes

_C = 64
_NCHUNKS = N_EDGES // _C
_ROUNDS = -(-_NCHUNKS // _NW)
_DBLK = 16


def _make_kernel():
    mesh = plsc.VectorSubcoreMesh(core_axis_name="c", subcore_axis_name="s")

    @functools.partial(
        pl.kernel,
        mesh=mesh,
        out_type=jax.ShapeDtypeStruct((N_EDGES,), jnp.float32),
        scratch_types=[
            pltpu.VMEM((2, _C), jnp.int32),
            pltpu.VMEM((2, _C), jnp.int32),
            pltpu.VMEM((_C, D_FEAT), jnp.float32),
            pltpu.VMEM((_C, D_FEAT), jnp.float32),
            pltpu.VMEM((_C, D_FEAT), jnp.float32),
            pltpu.VMEM((_C, D_FEAT), jnp.float32),
            pltpu.VMEM((_C,), jnp.float32),
            pltpu.VMEM((_L, 17), jnp.float32),
            pltpu.VMEM_SHARED((N_NODES, D_FEAT), jnp.float32),
            pltpu.SemaphoreType.DMA,
            pltpu.SemaphoreType.DMA,
            pltpu.SemaphoreType.DMA,
            pltpu.SemaphoreType.DMA,
        ],
        compiler_params=pltpu.CompilerParams(needs_layout_passes=False),
    )
    def dot_scores(idx_hbm, h_hbm, out_hbm,
                   i0, i1, u0, u1, v0, v1, scores, pmat, h_sp,
                   su0, su1, sv0, sv1):
        wid = lax.axis_index("s") * _NC + lax.axis_index("c")
        sid = lax.axis_index("s")
        lanes = lax.iota(jnp.int32, _L)

        idx_slot = [i0, i1]
        u_slot = [u0, u1]
        v_slot = [v0, v1]
        su_slot = [su0, su1]
        sv_slot = [sv0, sv1]

        rows_per_tile = 624
        stage0 = sid * rows_per_tile
        pltpu.sync_copy(h_hbm.at[pl.ds(stage0, rows_per_tile)],
                        h_sp.at[pl.ds(stage0, rows_per_tile)])

        @pl.when(sid == 0)
        def _():
            tail = N_NODES - _NS * rows_per_tile
            pltpu.sync_copy(h_hbm.at[pl.ds(_NS * rows_per_tile, tail)],
                            h_sp.at[pl.ds(_NS * rows_per_tile, tail)])

        plsc.subcore_barrier()

        def issue(t, slot):
            cid = t * _NW + wid

            @pl.when(cid < _NCHUNKS)
            def _():
                pltpu.sync_copy(idx_hbm.at[cid], idx_slot[slot])
                pltpu.async_copy(h_sp.at[idx_slot[slot].at[0]],
                                 u_slot[slot], su_slot[slot])
                pltpu.async_copy(h_sp.at[idx_slot[slot].at[1]],
                                 v_slot[slot], sv_slot[slot])

        def consume(t, slot):
            cid = t * _NW + wid

            @pl.when(cid < _NCHUNKS)
            def _():
                u_rows = u_slot[slot]
                v_rows = v_slot[slot]
                pltpu.make_async_copy(h_sp.at[idx_slot[slot].at[0]],
                                      u_rows, su_slot[slot]).wait()
                pltpu.make_async_copy(h_sp.at[idx_slot[slot].at[1]],
                                      v_rows, sv_slot[slot]).wait()

                def group_body(g, gcarry):
                    rows = g * _L + lanes
                    for j in range(_L):
                        e = g * _L + j
                        parts = []
                        for k in range(D_FEAT // _L):
                            uu = u_rows[e, pl.ds(k * _L, _L)]
                            vv = v_rows[e, pl.ds(k * _L, _L)]
                            parts.append(uu * vv)
                        while len(parts) > 1:
                            parts = [parts[i] + parts[i + 1]
                                     for i in range(0, len(parts), 2)]
                        pmat[j, pl.ds(0, _L)] = parts[0]
                    acc = jnp.zeros((_L,), jnp.float32)
                    for k in range(_L):
                        col = jnp.full((_L,), k, jnp.int32)
                        acc = acc + plsc.load_gather(pmat, [lanes, col])
                    plsc.store_scatter(scores, [rows], acc)
                    return gcarry

                lax.fori_loop(0, _C // _L, group_body, 0)
                pltpu.sync_copy(scores, out_hbm.at[pl.ds(cid * _C, _C)])

        issue(jnp.int32(0), 0)

        def pair_body(p, carry):
            t0 = p * 2
            issue(t0 + 1, 1)
            consume(t0, 0)
            issue(t0 + 2, 0)
            consume(t0 + 1, 1)
            return carry

        lax.fori_loop(0, (_ROUNDS + 1) // 2, pair_body, 0)

    return dot_scores


_dot_scores = _make_kernel()


def kernel(h, edge_index):
    idx = edge_index.astype(jnp.int32).reshape(2, _NCHUNKS, _C)
    idx_packed = idx.transpose(1, 0, 2)
    return _dot_scores(idx_packed, h)


# speedup vs baseline: 6.0641x; 1.3017x over previous
import functools

import jax
import jax.numpy as jnp
from jax import lax
from jax.experimental import pallas as pl
from jax.experimental.pallas import tpu as pltpu
from jax.experimental.pallas import tpu_sc as plsc

N_NODES = 10000
N_EDGES = 320000
D_FEAT = 128

_INFO = plsc.get_sparse_core_info()
_NC = _INFO.num_cores
_NS = _INFO.num_subcores
_NW = _NC * _NS
_L = _INFO.num_lanes

_C = 64
_NCHUNKS = N_EDGES // _C
_ROUNDS = -(-_NCHUNKS // _NW)


def _make_kernel():
    mesh = plsc.VectorSubcoreMesh(core_axis_name="c", subcore_axis_name="s")

    @functools.partial(
        pl.kernel,
        mesh=mesh,
        out_type=jax.ShapeDtypeStruct((N_EDGES,), jnp.float32),
        scratch_types=[
            pltpu.VMEM((2, _C), jnp.int32),
            pltpu.VMEM((2, _C), jnp.int32),
            pltpu.VMEM((_C, D_FEAT), jnp.float32),
            pltpu.VMEM((_C, D_FEAT), jnp.float32),
            pltpu.VMEM((_C, D_FEAT), jnp.float32),
            pltpu.VMEM((_C, D_FEAT), jnp.float32),
            pltpu.VMEM((_C,), jnp.float32),
            pltpu.VMEM((_C,), jnp.float32),
            pltpu.VMEM((_L, 17), jnp.float32),
            pltpu.VMEM_SHARED((N_NODES, D_FEAT), jnp.float32),
            pltpu.SemaphoreType.DMA,
            pltpu.SemaphoreType.DMA,
            pltpu.SemaphoreType.DMA,
            pltpu.SemaphoreType.DMA,
            pltpu.SemaphoreType.DMA,
            pltpu.SemaphoreType.DMA,
            pltpu.SemaphoreType.DMA,
            pltpu.SemaphoreType.DMA,
        ],
        compiler_params=pltpu.CompilerParams(needs_layout_passes=False),
    )
    def dot_scores(idx_hbm, h_hbm, out_hbm,
                   i0, i1, u0, u1, v0, v1, s0, s1, pmat, h_sp,
                   su0, su1, sv0, sv1, si0, si1, so0, so1):
        wid = lax.axis_index("s") * _NC + lax.axis_index("c")
        sid = lax.axis_index("s")
        lanes = lax.iota(jnp.int32, _L)

        idx_slot = [i0, i1]
        u_slot = [u0, u1]
        v_slot = [v0, v1]
        sc_slot = [s0, s1]
        su_slot = [su0, su1]
        sv_slot = [sv0, sv1]
        si_slot = [si0, si1]
        so_slot = [so0, so1]

        rows_per_tile = 624
        stage0 = sid * rows_per_tile
        pltpu.sync_copy(h_hbm.at[pl.ds(stage0, rows_per_tile)],
                        h_sp.at[pl.ds(stage0, rows_per_tile)])

        @pl.when(sid == 0)
        def _():
            tail = N_NODES - _NS * rows_per_tile
            pltpu.sync_copy(h_hbm.at[pl.ds(_NS * rows_per_tile, tail)],
                            h_sp.at[pl.ds(_NS * rows_per_tile, tail)])

        plsc.subcore_barrier()

        def cid_of(t):
            return t * _NW + wid

        def idx_copy(t, slot):
            @pl.when(cid_of(t) < _NCHUNKS)
            def _():
                pltpu.async_copy(idx_hbm.at[cid_of(t)], idx_slot[slot],
                                 si_slot[slot])

        def idx_wait(t, slot):
            @pl.when(cid_of(t) < _NCHUNKS)
            def _():
                pltpu.make_async_copy(idx_hbm.at[cid_of(t)], idx_slot[slot],
                                      si_slot[slot]).wait()

        def gathers_issue(t, slot):
            @pl.when(cid_of(t) < _NCHUNKS)
            def _():
                pltpu.async_copy(h_sp.at[idx_slot[slot].at[0]],
                                 u_slot[slot], su_slot[slot])
                pltpu.async_copy(h_sp.at[idx_slot[slot].at[1]],
                                 v_slot[slot], sv_slot[slot])

        def gathers_wait(t, slot):
            @pl.when(cid_of(t) < _NCHUNKS)
            def _():
                pltpu.make_async_copy(h_sp.at[idx_slot[slot].at[0]],
                                      u_slot[slot], su_slot[slot]).wait()
                pltpu.make_async_copy(h_sp.at[idx_slot[slot].at[1]],
                                      v_slot[slot], sv_slot[slot]).wait()

        def out_drain(slot):
            pltpu.make_async_copy(sc_slot[slot], out_hbm.at[pl.ds(0, _C)],
                                  so_slot[slot]).wait()

        def compute(t, slot):
            cid = cid_of(t)

            @pl.when(cid < _NCHUNKS)
            def _():
                u_rows = u_slot[slot]
                v_rows = v_slot[slot]
                scores = sc_slot[slot]

                @pl.when(t >= 2)
                def _():
                    out_drain(slot)

                def group_body(g, gcarry):
                    rows = g * _L + lanes
                    for j in range(_L):
                        e = g * _L + j
                        parts = []
                        for k in range(D_FEAT // _L):
                            uu = u_rows[e, pl.ds(k * _L, _L)]
                            vv = v_rows[e, pl.ds(k * _L, _L)]
                            parts.append(uu * vv)
                        while len(parts) > 1:
                            parts = [parts[i] + parts[i + 1]
                                     for i in range(0, len(parts), 2)]
                        pmat[j, pl.ds(0, _L)] = parts[0]
                    acc = jnp.zeros((_L,), jnp.float32)
                    for k in range(_L):
                        col = jnp.full((_L,), k, jnp.int32)
                        acc = acc + plsc.load_gather(pmat, [lanes, col])
                    plsc.store_scatter(scores, [rows], acc)
                    return gcarry

                lax.fori_loop(0, _C // _L, group_body, 0)
                pltpu.async_copy(scores, out_hbm.at[pl.ds(cid * _C, _C)],
                                 so_slot[slot])

        def step(t, slot):
            other = 1 - slot
            idx_wait(t + 1, other)
            gathers_issue(t + 1, other)
            gathers_wait(t, slot)
            idx_copy(t + 2, slot)
            compute(t, slot)

        idx_copy(jnp.int32(0), 0)
        idx_wait(jnp.int32(0), 0)
        gathers_issue(jnp.int32(0), 0)
        idx_copy(jnp.int32(1), 1)

        def pair_body(p, carry):
            t0 = p * 2
            step(t0, 0)
            step(t0 + 1, 1)
            return carry

        lax.fori_loop(0, (_ROUNDS + 1) // 2, pair_body, 0)

        n_valid = (_NCHUNKS - wid + _NW - 1) // _NW

        for s in (0, 1):
            @pl.when(n_valid > s)
            def _(s=s):
                out_drain(s)

    return dot_scores


_dot_scores = _make_kernel()


def kernel(h, edge_index):
    idx = edge_index.astype(jnp.int32).reshape(2, _NCHUNKS, _C)
    idx_packed = idx.transpose(1, 0, 2)
    return _dot_scores(idx_packed, h)


# speedup vs baseline: 7.9805x; 1.3160x over previous
import functools

import jax
import jax.numpy as jnp
from jax import lax
from jax.experimental import pallas as pl
from jax.experimental.pallas import tpu as pltpu
from jax.experimental.pallas import tpu_sc as plsc

N_NODES = 10000
N_EDGES = 320000
D_FEAT = 128

_INFO = plsc.get_sparse_core_info()
_NC = _INFO.num_cores
_NS = _INFO.num_subcores
_NW = _NC * _NS
_L = _INFO.num_lanes

_C = 64
_NCHUNKS = N_EDGES // _C
_ROUNDS = -(-_NCHUNKS // _NW)


def _make_kernel():
    mesh = plsc.VectorSubcoreMesh(core_axis_name="c", subcore_axis_name="s")

    @functools.partial(
        pl.kernel,
        mesh=mesh,
        out_type=jax.ShapeDtypeStruct((N_EDGES,), jnp.float32),
        scratch_types=[
            pltpu.VMEM((2, _C), jnp.int32),
            pltpu.VMEM((2, _C), jnp.int32),
            pltpu.VMEM((_C, D_FEAT), jnp.float32),
            pltpu.VMEM((_C, D_FEAT), jnp.float32),
            pltpu.VMEM((_C, D_FEAT), jnp.float32),
            pltpu.VMEM((_C, D_FEAT), jnp.float32),
            pltpu.VMEM((_C,), jnp.float32),
            pltpu.VMEM((_C,), jnp.float32),
            pltpu.VMEM((_L, 17), jnp.float32),
            pltpu.VMEM_SHARED((N_NODES, D_FEAT), jnp.float32),
            pltpu.SemaphoreType.DMA,
            pltpu.SemaphoreType.DMA,
            pltpu.SemaphoreType.DMA,
            pltpu.SemaphoreType.DMA,
            pltpu.SemaphoreType.DMA,
            pltpu.SemaphoreType.DMA,
            pltpu.SemaphoreType.DMA,
            pltpu.SemaphoreType.DMA,
        ],
        compiler_params=pltpu.CompilerParams(needs_layout_passes=False),
    )
    def dot_scores(idx_hbm, h_hbm, out_hbm,
                   i0, i1, u0, u1, v0, v1, s0, s1, pmat, h_sp,
                   su0, su1, sv0, sv1, si0, si1, so0, so1):
        wid = lax.axis_index("s") * _NC + lax.axis_index("c")
        sid = lax.axis_index("s")
        lanes = lax.iota(jnp.int32, _L)

        idx_slot = [i0, i1]
        u_slot = [u0, u1]
        v_slot = [v0, v1]
        sc_slot = [s0, s1]
        su_slot = [su0, su1]
        sv_slot = [sv0, sv1]
        si_slot = [si0, si1]
        so_slot = [so0, so1]

        rows_per_tile = 624
        stage0 = sid * rows_per_tile
        pltpu.sync_copy(h_hbm.at[pl.ds(stage0, rows_per_tile)],
                        h_sp.at[pl.ds(stage0, rows_per_tile)])

        @pl.when(sid == 0)
        def _():
            tail = N_NODES - _NS * rows_per_tile
            pltpu.sync_copy(h_hbm.at[pl.ds(_NS * rows_per_tile, tail)],
                            h_sp.at[pl.ds(_NS * rows_per_tile, tail)])

        plsc.subcore_barrier()

        def cid_of(t):
            return t * _NW + wid

        def idx_copy(t, slot):
            @pl.when(cid_of(t) < _NCHUNKS)
            def _():
                pltpu.async_copy(idx_hbm.at[cid_of(t)], idx_slot[slot],
                                 si_slot[slot])

        def idx_wait(t, slot):
            @pl.when(cid_of(t) < _NCHUNKS)
            def _():
                pltpu.make_async_copy(idx_hbm.at[cid_of(t)], idx_slot[slot],
                                      si_slot[slot]).wait()

        def gathers_issue(t, slot):
            @pl.when(cid_of(t) < _NCHUNKS)
            def _():
                pltpu.async_copy(h_sp.at[idx_slot[slot].at[0]],
                                 u_slot[slot], su_slot[slot])
                pltpu.async_copy(h_sp.at[idx_slot[slot].at[1]],
                                 v_slot[slot], sv_slot[slot])

        def gathers_wait(t, slot):
            @pl.when(cid_of(t) < _NCHUNKS)
            def _():
                pltpu.make_async_copy(h_sp.at[idx_slot[slot].at[0]],
                                      u_slot[slot], su_slot[slot]).wait()
                pltpu.make_async_copy(h_sp.at[idx_slot[slot].at[1]],
                                      v_slot[slot], sv_slot[slot]).wait()

        def out_drain(slot):
            pltpu.make_async_copy(sc_slot[slot], out_hbm.at[pl.ds(0, _C)],
                                  so_slot[slot]).wait()

        def compute(t, slot):
            cid = cid_of(t)

            @pl.when(cid < _NCHUNKS)
            def _():
                u_rows = u_slot[slot]
                v_rows = v_slot[slot]
                scores = sc_slot[slot]

                @pl.when(t >= 2)
                def _():
                    out_drain(slot)

                def edge_loads(e):
                    us = [u_rows[e, pl.ds(k * _L, _L)]
                          for k in range(D_FEAT // _L)]
                    vs = [v_rows[e, pl.ds(k * _L, _L)]
                          for k in range(D_FEAT // _L)]
                    return us, vs

                def edge_arith(j, us, vs):
                    parts = [us[k] * vs[k] for k in range(D_FEAT // _L)]
                    while len(parts) > 1:
                        parts = [parts[i] + parts[i + 1]
                                 for i in range(0, len(parts), 2)]
                    pmat[j, pl.ds(0, _L)] = parts[0]

                def group_body(g, gcarry):
                    rows = g * _L + lanes
                    prev = edge_loads(g * _L)
                    for j in range(_L):
                        cur = edge_loads(g * _L + j + 1) if j + 1 < _L else None
                        edge_arith(j, *prev)
                        prev = cur
                    gath = [plsc.load_gather(
                                pmat, [lanes, jnp.full((_L,), k, jnp.int32)])
                            for k in range(_L)]
                    while len(gath) > 1:
                        gath = [gath[i] + gath[i + 1]
                                for i in range(0, len(gath), 2)]
                    plsc.store_scatter(scores, [rows], gath[0])
                    return gcarry

                lax.fori_loop(0, _C // _L, group_body, 0)
                pltpu.async_copy(scores, out_hbm.at[pl.ds(cid * _C, _C)],
                                 so_slot[slot])

        def step(t, slot):
            other = 1 - slot
            idx_wait(t + 1, other)
            gathers_issue(t + 1, other)
            gathers_wait(t, slot)
            idx_copy(t + 2, slot)
            compute(t, slot)

        idx_copy(jnp.int32(0), 0)
        idx_wait(jnp.int32(0), 0)
        gathers_issue(jnp.int32(0), 0)
        idx_copy(jnp.int32(1), 1)

        def pair_body(p, carry):
            t0 = p * 2
            step(t0, 0)
            step(t0 + 1, 1)
            return carry

        lax.fori_loop(0, (_ROUNDS + 1) // 2, pair_body, 0)

        n_valid = (_NCHUNKS - wid + _NW - 1) // _NW

        for s in (0, 1):
            @pl.when(n_valid > s)
            def _(s=s):
                out_drain(s)

    return dot_scores


_dot_scores = _make_kernel()


def kernel(h, edge_index):
    idx = edge_index.astype(jnp.int32).reshape(2, _NCHUNKS, _C)
    idx_packed = idx.transpose(1, 0, 2)
    return _dot_scores(idx_packed, h)
